# baseline (speedup 1.0000x reference)
import math

import jax
import jax.numpy as jnp
from jax import lax
from jax.experimental import pallas as pl
from jax.experimental.pallas import tpu as pltpu
from jax.experimental.pallas import tpu_sc as plsc

EMB_D = 32
SCALE = math.sqrt(float(EMB_D))

NC, NS, LANES = 2, 16, 16
NW = NC * NS

N_SEQ, N_TOK = 16384, 50
B_TOTAL = N_SEQ * N_TOK
GIDX = 128
NBLK = B_TOTAL // GIDX
BPW = NBLK // NW
SB = N_SEQ // GIDX
TILE_W = 8 * GIDX
CPB = 4
NCH = BPW // CPB
CH_W = CPB * TILE_W
UNROLL = 8


def _emb_body(idx_hbm, table_hbm, out_hbm,
              idx_v, rows0, rows1, t40, t41, gsem0, gsem1, ssem0, ssem1):
    wid = lax.axis_index("s") * NC + lax.axis_index("c")
    g0 = wid * BPW

    pltpu.sync_copy(idx_hbm.at[pl.ds(g0, BPW)], idx_v)

    lane = lax.iota(jnp.int32, LANES)
    pat_lo = [(lane >> 3) * CH_W + (lane & 7) * GIDX + r
              for r in range(UNROLL)]
    pat_hi = [p + 2 * CH_W for p in pat_lo]

    def issue_gathers(c, rows_b, gsem_b):
        for q in range(CPB):
            pltpu.async_copy(table_hbm.at[idx_v.at[c * CPB + q]],
                             rows_b.at[pl.ds(q * GIDX, GIDX)], gsem_b)

    def wait_gathers(c, rows_b, gsem_b):
        for q in range(CPB):
            pltpu.make_async_copy(table_hbm.at[idx_v.at[c * CPB + q]],
                                  rows_b.at[pl.ds(q * GIDX, GIDX)],
                                  gsem_b).wait()

    def transpose_scale(rows_b, t4_b):
        for q in range(CPB):
            def body(it, _, q=q):
                sp0 = it * UNROLL
                spv = jnp.full((LANES,), q * TILE_W + sp0, dtype=jnp.int32)
                for r in range(UNROLL):
                    row = q * GIDX + sp0 + r
                    lo = rows_b[row, pl.ds(0, LANES)] * SCALE
                    hi = rows_b[row, pl.ds(LANES, LANES)] * SCALE
                    plsc.store_scatter(t4_b, [pat_lo[r] + spv], lo)
                    plsc.store_scatter(t4_b, [pat_hi[r] + spv], hi)
                return 0

            lax.fori_loop(0, GIDX // UNROLL, body, 0)

    def issue_stores(j, stg, t4_b, ssem_b):
        for dt in range(4):
            pltpu.async_copy(t4_b.at[pl.ds(dt * CH_W, CH_W)],
                             out_hbm.at[j, dt, stg], ssem_b)

    def wait_stores(j, stg, t4_b, ssem_b):
        for dt in range(4):
            pltpu.make_async_copy(t4_b.at[pl.ds(dt * CH_W, CH_W)],
                                  out_hbm.at[j, dt, stg], ssem_b).wait()

    issue_gathers(0, rows0, gsem0)
    issue_gathers(1, rows1, gsem1)

    bufs = ((rows0, t40, gsem0, ssem0), (rows1, t41, gsem1, ssem1))

    def pair(t, _):
        for b in range(2):
            rows_b, t4_b, gsem_b, ssem_b = bufs[b]
            c = 2 * t + b
            g = g0 + c * CPB
            j = g >> 7
            stg = (g & (SB - 1)) >> 2
            wait_gathers(c, rows_b, gsem_b)

            @pl.when(t > 0)
            def _():
                gp = g0 + (c - 2) * CPB
                wait_stores(gp >> 7, (gp & (SB - 1)) >> 2, t4_b, ssem_b)

            transpose_scale(rows_b, t4_b)

            @pl.when(c + 2 < NCH)
            def _():
                issue_gathers(c + 2, rows_b, gsem_b)

            issue_stores(j, stg, t4_b, ssem_b)
        return 0

    lax.fori_loop(0, NCH // 2, pair, 0)

    for b in range(2):
        rows_b, t4_b, gsem_b, ssem_b = bufs[b]
        g = g0 + (NCH - 2 + b) * CPB
        wait_stores(g >> 7, (g & (SB - 1)) >> 2, t4_b, ssem_b)


@jax.jit
def _emb(idx2, table):
    mesh = plsc.VectorSubcoreMesh(core_axis_name="c", subcore_axis_name="s")
    f = pl.kernel(
        _emb_body,
        out_type=jax.ShapeDtypeStruct((N_TOK, 4, SB // CPB, CH_W),
                                      jnp.float32),
        mesh=mesh,
        scratch_types=[
            pltpu.VMEM((BPW, GIDX), jnp.int32),
            pltpu.VMEM((CPB * GIDX, EMB_D), jnp.float32),
            pltpu.VMEM((CPB * GIDX, EMB_D), jnp.float32),
            pltpu.VMEM((4 * CH_W,), jnp.float32),
            pltpu.VMEM((4 * CH_W,), jnp.float32),
            pltpu.SemaphoreType.DMA,
            pltpu.SemaphoreType.DMA,
            pltpu.SemaphoreType.DMA,
            pltpu.SemaphoreType.DMA,
        ],
        compiler_params=pltpu.CompilerParams(
            use_tc_tiling_on_sc=False, needs_layout_passes=False),
    )
    return f(idx2, table)


def kernel(x, embed_table):
    idx2 = x.T.reshape(NBLK, GIDX).astype(jnp.int32)
    out4 = _emb(idx2, embed_table)
    return (out4.reshape(N_TOK, 4, SB // CPB, CPB, 8, GIDX)
            .transpose(2, 3, 5, 0, 1, 4)
            .reshape(N_SEQ, N_TOK, EMB_D))


# speedup vs baseline: 1.3518x; 1.3518x over previous
import math

import jax
import jax.numpy as jnp
from jax import lax
from jax.experimental import pallas as pl
from jax.experimental.pallas import tpu as pltpu
from jax.experimental.pallas import tpu_sc as plsc

EMB_D = 32
SCALE = math.sqrt(float(EMB_D))

NC, NS, LANES = 2, 16, 16
NW = NC * NS

N_SEQ, N_TOK = 16384, 50
B_TOTAL = N_SEQ * N_TOK
GIDX = 128
NBLK = B_TOTAL // GIDX
BPW = NBLK // NW
SB = N_SEQ // GIDX
TILE_W = 8 * GIDX
CPB = 4
NCH = BPW // CPB
CH_W = CPB * TILE_W
UNROLL = 8


def _emb_body(idx_hbm, table_hbm, out_hbm,
              idx_v, rows0, rows1, t40, t41, gsem0, gsem1, ssem0, ssem1):
    wid = lax.axis_index("s") * NC + lax.axis_index("c")
    g0 = wid * BPW

    pltpu.sync_copy(idx_hbm.at[pl.ds(g0, BPW)], idx_v)

    lane = lax.iota(jnp.int32, LANES)
    mv = [(lane + k) & (LANES - 1) for k in range(LANES)]
    cvec = [[m + d0 for m in mv] for d0 in (0, LANES)]
    dvec = [(m >> 3) * CH_W + (m & 7) * GIDX + lane for m in mv]

    def issue_gathers(c, rows_b, gsem_b):
        for q in range(CPB):
            pltpu.async_copy(table_hbm.at[idx_v.at[c * CPB + q]],
                             rows_b.at[pl.ds(q * GIDX, GIDX)], gsem_b)

    def wait_gathers(c, rows_b, gsem_b):
        for q in range(CPB):
            pltpu.make_async_copy(table_hbm.at[idx_v.at[c * CPB + q]],
                                  rows_b.at[pl.ds(q * GIDX, GIDX)],
                                  gsem_b).wait()

    def transpose_scale(rows_b, t4_b):
        def body(it, _):
            sp0 = it * LANES
            for q in range(CPB):
                rvec = lane + (q * GIDX + sp0)
                for h in range(2):
                    base = h * 2 * CH_W + q * TILE_W + sp0
                    bvec = jnp.full((LANES,), base, dtype=jnp.int32)
                    for k in range(LANES):
                        v = plsc.load_gather(rows_b, [rvec, cvec[h][k]])
                        plsc.store_scatter(t4_b, [dvec[k] + bvec], v * SCALE)
            return 0

        lax.fori_loop(0, GIDX // LANES, body, 0)

    def issue_stores(j, stg, t4_b, ssem_b):
        for dt in range(4):
            pltpu.async_copy(t4_b.at[pl.ds(dt * CH_W, CH_W)],
                             out_hbm.at[j, dt, stg], ssem_b)

    def wait_stores(j, stg, t4_b, ssem_b):
        for dt in range(4):
            pltpu.make_async_copy(t4_b.at[pl.ds(dt * CH_W, CH_W)],
                                  out_hbm.at[j, dt, stg], ssem_b).wait()

    issue_gathers(0, rows0, gsem0)
    issue_gathers(1, rows1, gsem1)

    bufs = ((rows0, t40, gsem0, ssem0), (rows1, t41, gsem1, ssem1))

    def pair(t, _):
        for b in range(2):
            rows_b, t4_b, gsem_b, ssem_b = bufs[b]
            c = 2 * t + b
            g = g0 + c * CPB
            j = g >> 7
            stg = (g & (SB - 1)) >> 2
            wait_gathers(c, rows_b, gsem_b)

            @pl.when(t > 0)
            def _():
                gp = g0 + (c - 2) * CPB
                wait_stores(gp >> 7, (gp & (SB - 1)) >> 2, t4_b, ssem_b)

            transpose_scale(rows_b, t4_b)

            @pl.when(c + 2 < NCH)
            def _():
                issue_gathers(c + 2, rows_b, gsem_b)

            issue_stores(j, stg, t4_b, ssem_b)
        return 0

    lax.fori_loop(0, NCH // 2, pair, 0)

    for b in range(2):
        rows_b, t4_b, gsem_b, ssem_b = bufs[b]
        g = g0 + (NCH - 2 + b) * CPB
        wait_stores(g >> 7, (g & (SB - 1)) >> 2, t4_b, ssem_b)


@jax.jit
def _emb(idx2, table):
    mesh = plsc.VectorSubcoreMesh(core_axis_name="c", subcore_axis_name="s")
    f = pl.kernel(
        _emb_body,
        out_type=jax.ShapeDtypeStruct((N_TOK, 4, SB // CPB, CH_W),
                                      jnp.float32),
        mesh=mesh,
        scratch_types=[
            pltpu.VMEM((BPW, GIDX), jnp.int32),
            pltpu.VMEM((CPB * GIDX, EMB_D), jnp.float32),
            pltpu.VMEM((CPB * GIDX, EMB_D), jnp.float32),
            pltpu.VMEM((4 * CH_W,), jnp.float32),
            pltpu.VMEM((4 * CH_W,), jnp.float32),
            pltpu.SemaphoreType.DMA,
            pltpu.SemaphoreType.DMA,
            pltpu.SemaphoreType.DMA,
            pltpu.SemaphoreType.DMA,
        ],
        compiler_params=pltpu.CompilerParams(
            use_tc_tiling_on_sc=False, needs_layout_passes=False),
    )
    return f(idx2, table)


def kernel(x, embed_table):
    idx2 = x.T.reshape(NBLK, GIDX).astype(jnp.int32)
    out4 = _emb(idx2, embed_table)
    return (out4.reshape(N_TOK, 4, SB // CPB, CPB, 8, GIDX)
            .transpose(2, 3, 5, 0, 1, 4)
            .reshape(N_SEQ, N_TOK, EMB_D))
